# baseline (speedup 1.0000x reference)
import jax
import jax.numpy as jnp
from jax import lax
from jax.experimental import pallas as pl
from jax.experimental.pallas import tpu as pltpu
from jax.experimental.pallas import tpu_sc as plsc

N = 10000
E = 160000
FIN = 256
H = 512
NP = 10240
NTILE = 16
EPT = E // NTILE
EPV = EPT // 16
BE = 80
NBLK = EPT // BE
CHUNK = 128
NCH = H // CHUNK
NROWT = NP // NTILE
ZROWS = 32
NLAST = N - (NTILE - 1) * NROWT


def _zero_vmem_2d(ref, nrows, ncols):
    zeros16 = jnp.zeros((16,), jnp.float32)

    def body(r, _):
        for k in range(ncols // 16):
            ref[r, pl.ds(k * 16, 16)] = zeros16
        return 0

    lax.fori_loop(0, nrows, body, 0)


def _edge_stats_body(dst_hbm, ts_hbm, nt_out, deg_out,
                     dst_v, val_v, acc_v, tmp_v, red_v, shared):
    cid = lax.axis_index("c")
    tid = lax.axis_index("s")
    zeros16 = jnp.zeros((16,), jnp.float32)
    ones16 = jnp.ones((16,), jnp.float32)

    def zbody(i, _):
        acc_v[pl.ds(i * 16, 16)] = zeros16
        return 0
    lax.fori_loop(0, NP // 16, zbody, 0)

    pltpu.sync_copy(dst_hbm.at[pl.ds(tid * EPT, EPT)], dst_v)
    pltpu.sync_copy(ts_hbm.at[pl.ds(tid * EPT, EPT)], val_v)

    @pl.when(cid == 0)
    def _():
        iota = lax.iota(jnp.int32, 16)

        def process(i, _):
            idx = dst_v[pl.ds(i * 16, 16)]
            tv = val_v[pl.ds(i * 16, 16)]
            for r in range(1, 16):
                perm = (iota + r) & 15
                dr = idx.at[perm].get(mode="promise_in_bounds")
                tr = tv.at[perm].get(mode="promise_in_bounds")
                tv = jnp.where(dr == idx, jnp.maximum(tv, tr), tv)
            cur = plsc.load_gather(acc_v, [idx])
            plsc.store_scatter(acc_v, [idx], jnp.maximum(cur, tv))
            return 0

        lax.fori_loop(0, EPV, process, 0)

    @pl.when(cid == 1)
    def _():
        def process(i, _):
            idx = dst_v[pl.ds(i * 16, 16)]
            plsc.addupdate_scatter(acc_v, [idx], ones16)
            return 0

        lax.fori_loop(0, EPV, process, 0)

    pltpu.sync_copy(acc_v, shared.at[tid])
    plsc.subcore_barrier()
    pltpu.sync_copy(shared.at[:, pl.ds(tid * 640, 640)], tmp_v)

    is_max = cid == 0

    def rbody(g, _):
        r = tmp_v[0, pl.ds(g * 16, 16)]
        for k in range(1, NTILE):
            v = tmp_v[k, pl.ds(g * 16, 16)]
            r = jnp.where(is_max, jnp.maximum(r, v), r + v)
        red_v[pl.ds(g * 16, 16)] = r
        return 0

    lax.fori_loop(0, 640 // 16, rbody, 0)

    @pl.when(cid == 0)
    def _():
        pltpu.sync_copy(red_v, nt_out.at[pl.ds(tid * 640, 640)])

    @pl.when(cid == 1)
    def _():
        pltpu.sync_copy(red_v, deg_out.at[pl.ds(tid * 640, 640)])


@jax.jit
def _edge_stats(dst, ts):
    mesh = plsc.VectorSubcoreMesh(core_axis_name="c", subcore_axis_name="s")
    return pl.kernel(
        _edge_stats_body,
        out_type=[jax.ShapeDtypeStruct((NP,), jnp.float32),
                  jax.ShapeDtypeStruct((NP,), jnp.float32)],
        mesh=mesh,
        scratch_types=[
            pltpu.VMEM((EPT,), jnp.int32),
            pltpu.VMEM((EPT,), jnp.float32),
            pltpu.VMEM((NP,), jnp.float32),
            pltpu.VMEM((NTILE, 640), jnp.float32),
            pltpu.VMEM((640,), jnp.float32),
            pltpu.VMEM_SHARED((NTILE, NP), jnp.float32),
        ],
        compiler_params=pltpu.CompilerParams(needs_layout_passes=False),
    )(dst, ts)


def _agg_body(src_hbm, dst_hbm, y_hbm, agg_out,
              src_v, dst_v, rows0, rows1, zero_v, acc_sh,
              sem0, sem1):
    cid = lax.axis_index("c")
    tid = lax.axis_index("s")

    pltpu.sync_copy(src_hbm.at[pl.ds(tid * EPT, EPT)], src_v)
    pltpu.sync_copy(dst_hbm.at[pl.ds(tid * EPT, EPT)], dst_v)
    _zero_vmem_2d(zero_v, ZROWS, CHUNK)

    def process(j, this_rows, this_sem, issue_j, next_rows, next_sem):
        if issue_j is not None:
            pltpu.async_copy(y_hbm.at[src_v.at[pl.ds(issue_j * BE, BE)]],
                             next_rows, next_sem)
        pltpu.make_async_copy(y_hbm.at[src_v.at[pl.ds(j * BE, BE)]],
                              this_rows, this_sem).wait()
        pltpu.sync_copy(this_rows, acc_sh.at[dst_v.at[pl.ds(j * BE, BE)]],
                        add=True)

    for ci in range(NCH // 2):
        def gbody(r, _):
            for k in range(BE // 16):
                off = r * BE + k * 16
                v = src_v[pl.ds(off, 16)]
                if ci == 0:
                    v = v * NCH + cid * (NCH // 2)
                else:
                    v = v + 1
                src_v[pl.ds(off, 16)] = v
            return 0
        lax.fori_loop(0, NBLK, gbody, 0)

        for z in range(NROWT // ZROWS):
            pltpu.sync_copy(
                zero_v, acc_sh.at[pl.ds(tid * NROWT + z * ZROWS, ZROWS)])
        plsc.subcore_barrier()

        pltpu.async_copy(y_hbm.at[src_v.at[pl.ds(0, BE)]], rows0, sem0)

        def blk(jj, _):
            j0 = jj * 2
            process(j0, rows0, sem0, j0 + 1, rows1, sem1)
            process(j0 + 1, rows1, sem1, j0 + 2, rows0, sem0)
            return 0

        lax.fori_loop(0, (NBLK - 1) // 2, blk, 0)
        process(NBLK - 1, rows0, sem0, None, None, None)

        plsc.subcore_barrier()

        c = cid * (NCH // 2) + ci

        @pl.when(tid < NTILE - 1)
        def _():
            pltpu.sync_copy(acc_sh.at[pl.ds(tid * NROWT, NROWT)],
                            agg_out.at[c, pl.ds(tid * NROWT, NROWT)])

        @pl.when(tid == NTILE - 1)
        def _():
            pltpu.sync_copy(acc_sh.at[pl.ds((NTILE - 1) * NROWT, NLAST)],
                            agg_out.at[c, pl.ds((NTILE - 1) * NROWT, NLAST)])

        plsc.subcore_barrier()


@jax.jit
def _aggregate(src2, dst2, y2):
    mesh = plsc.VectorSubcoreMesh(core_axis_name="c", subcore_axis_name="s")
    return pl.kernel(
        _agg_body,
        out_type=jax.ShapeDtypeStruct((NCH, N, CHUNK), jnp.float32),
        mesh=mesh,
        scratch_types=[
            pltpu.VMEM((EPT,), jnp.int32),
            pltpu.VMEM((EPT,), jnp.int32),
            pltpu.VMEM((BE, CHUNK), jnp.float32),
            pltpu.VMEM((BE, CHUNK), jnp.float32),
            pltpu.VMEM((ZROWS, CHUNK), jnp.float32),
            pltpu.VMEM_SHARED((NP, CHUNK), jnp.float32),
            pltpu.SemaphoreType.DMA,
            pltpu.SemaphoreType.DMA,
        ],
        compiler_params=pltpu.CompilerParams(
            needs_layout_passes=False, use_tc_tiling_on_sc=False),
    )(src2, dst2, y2)


def _tc_a_body(x_ref, win_ref, bin_ref, wt_ref, bt_ref, ntc_ref, ntf_ref,
               deg_ref, wg_ref, y_ref):
    h = jnp.dot(x_ref[...], win_ref[...],
                preferred_element_type=jnp.float32) + bin_ref[...]
    ntf = ntf_ref[...]
    tmin = jnp.min(ntf)
    tmax = jnp.max(ntf)
    nt = ntc_ref[...]
    ntn = jnp.where(tmax > tmin, (nt - tmin) / (tmax - tmin + 1e-8), nt)
    h = h + ntn * wt_ref[...] + bt_ref[...]
    h = jnp.maximum(h, 0.0)
    dinv = lax.rsqrt(deg_ref[...] + 1.0)
    y_ref[...] = jnp.dot(h, wg_ref[...],
                         preferred_element_type=jnp.float32) * dinv


@jax.jit
def _tc_a(x, W_in, b_in, W_t, b_t, nt_col, nt_full, deg_col, W_gcn):
    blk = 400
    grid = N // blk
    return pl.pallas_call(
        _tc_a_body,
        grid=(grid,),
        in_specs=[
            pl.BlockSpec((blk, FIN), lambda i: (i, 0)),
            pl.BlockSpec((FIN, H), lambda i: (0, 0)),
            pl.BlockSpec((1, H), lambda i: (0, 0)),
            pl.BlockSpec((1, H), lambda i: (0, 0)),
            pl.BlockSpec((1, H), lambda i: (0, 0)),
            pl.BlockSpec((blk, 1), lambda i: (i, 0)),
            pl.BlockSpec((8, 1250), lambda i: (0, 0)),
            pl.BlockSpec((blk, 1), lambda i: (i, 0)),
            pl.BlockSpec((H, H), lambda i: (0, 0)),
        ],
        out_specs=pl.BlockSpec((blk, H), lambda i: (i, 0)),
        out_shape=jax.ShapeDtypeStruct((N, H), jnp.float32),
    )(x, W_in, b_in, W_t, b_t, nt_col, nt_full, deg_col, W_gcn)


def _tc_b_body(agg_ref, y_ref, deg_ref, bgcn_ref, wc1_ref, bc1_ref,
               wc2_ref, bc2_ref, out_ref):
    a = agg_ref[...]
    aggc = jnp.concatenate([a[c] for c in range(NCH)], axis=-1)
    dinv = lax.rsqrt(deg_ref[...] + 1.0)
    h2 = (aggc + y_ref[...]) * dinv + bgcn_ref[...]
    h1 = jnp.maximum(
        jnp.dot(h2, wc1_ref[...], preferred_element_type=jnp.float32)
        + bc1_ref[...], 0.0)
    out_ref[...] = jnp.dot(h1, wc2_ref[...],
                           preferred_element_type=jnp.float32) + bc2_ref[...]


@jax.jit
def _tc_b(agg, y, deg_col, b_gcn, W_c1, b_c1, W_c2, b_c2):
    blk = 400
    grid = N // blk
    return pl.pallas_call(
        _tc_b_body,
        grid=(grid,),
        in_specs=[
            pl.BlockSpec((NCH, blk, CHUNK), lambda i: (0, i, 0)),
            pl.BlockSpec((blk, H), lambda i: (i, 0)),
            pl.BlockSpec((blk, 1), lambda i: (i, 0)),
            pl.BlockSpec((1, H), lambda i: (0, 0)),
            pl.BlockSpec((H, 64), lambda i: (0, 0)),
            pl.BlockSpec((1, 64), lambda i: (0, 0)),
            pl.BlockSpec((64, 2), lambda i: (0, 0)),
            pl.BlockSpec((1, 2), lambda i: (0, 0)),
        ],
        out_specs=pl.BlockSpec((blk, 2), lambda i: (i, 0)),
        out_shape=jax.ShapeDtypeStruct((N, 2), jnp.float32),
    )(agg, y, deg_col, b_gcn, W_c1, b_c1, W_c2, b_c2)


def kernel(x, edge_index, timestamps, W_in, b_in, W_t, b_t, W_gcn, b_gcn,
           W_c1, b_c1, W_c2, b_c2):
    src = edge_index[0]
    dst = edge_index[1]
    nt_p, deg_p = _edge_stats(dst, timestamps)
    nt = nt_p[:N]
    deg_in = deg_p[:N]
    y = _tc_a(x, W_in, b_in.reshape(1, H), W_t, b_t.reshape(1, H),
              nt.reshape(N, 1), nt.reshape(8, 1250), deg_in.reshape(N, 1),
              W_gcn)
    agg = _aggregate(src, dst, y.reshape(N * NCH, CHUNK))
    return _tc_b(agg, y, deg_in.reshape(N, 1), b_gcn.reshape(1, H),
                 W_c1, b_c1.reshape(1, 64), W_c2, b_c2.reshape(1, 2))


# speedup vs baseline: 1.0505x; 1.0505x over previous
import jax
import jax.numpy as jnp
from jax import lax
from jax.experimental import pallas as pl
from jax.experimental.pallas import tpu as pltpu
from jax.experimental.pallas import tpu_sc as plsc

N = 10000
E = 160000
FIN = 256
H = 512
NP = 10240
NTILE = 16
EPT = E // NTILE
EPV = EPT // 16
BE = 80
NBLK = EPT // BE
CHUNK = 128
NCH = H // CHUNK
NROWT = NP // NTILE
ZROWS = 32
NLAST = N - (NTILE - 1) * NROWT


def _zero_vmem_2d(ref, nrows, ncols):
    zeros16 = jnp.zeros((16,), jnp.float32)

    def body(r, _):
        for k in range(ncols // 16):
            ref[r, pl.ds(k * 16, 16)] = zeros16
        return 0

    lax.fori_loop(0, nrows, body, 0)


def _edge_stats_body(dst_hbm, ts_hbm, nt_out, deg_out,
                     dst_v, val_v, acc_v, tmp_v, red_v, shared):
    cid = lax.axis_index("c")
    tid = lax.axis_index("s")
    zeros16 = jnp.zeros((16,), jnp.float32)
    ones16 = jnp.ones((16,), jnp.float32)

    def zbody(i, _):
        acc_v[pl.ds(i * 16, 16)] = zeros16
        return 0
    lax.fori_loop(0, NP // 16, zbody, 0)

    pltpu.sync_copy(dst_hbm.at[pl.ds(tid * EPT, EPT)], dst_v)
    pltpu.sync_copy(ts_hbm.at[pl.ds(tid * EPT, EPT)], val_v)

    @pl.when(cid == 0)
    def _():
        iota = lax.iota(jnp.int32, 16)

        def process(i, _):
            idx = dst_v[pl.ds(i * 16, 16)]
            tv = val_v[pl.ds(i * 16, 16)]
            for r in range(1, 16):
                perm = (iota + r) & 15
                dr = idx.at[perm].get(mode="promise_in_bounds")
                tr = tv.at[perm].get(mode="promise_in_bounds")
                tv = jnp.where(dr == idx, jnp.maximum(tv, tr), tv)
            cur = plsc.load_gather(acc_v, [idx])
            plsc.store_scatter(acc_v, [idx], jnp.maximum(cur, tv))
            return 0

        lax.fori_loop(0, EPV, process, 0)

    @pl.when(cid == 1)
    def _():
        def process(i, _):
            idx = dst_v[pl.ds(i * 16, 16)]
            plsc.addupdate_scatter(acc_v, [idx], ones16)
            return 0

        lax.fori_loop(0, EPV, process, 0)

    pltpu.sync_copy(acc_v, shared.at[tid])
    plsc.subcore_barrier()
    pltpu.sync_copy(shared.at[:, pl.ds(tid * 640, 640)], tmp_v)

    is_max = cid == 0

    def rbody(g, _):
        r = tmp_v[0, pl.ds(g * 16, 16)]
        for k in range(1, NTILE):
            v = tmp_v[k, pl.ds(g * 16, 16)]
            r = jnp.where(is_max, jnp.maximum(r, v), r + v)
        red_v[pl.ds(g * 16, 16)] = r
        return 0

    lax.fori_loop(0, 640 // 16, rbody, 0)

    @pl.when(cid == 0)
    def _():
        pltpu.sync_copy(red_v, nt_out.at[pl.ds(tid * 640, 640)])

    @pl.when(cid == 1)
    def _():
        pltpu.sync_copy(red_v, deg_out.at[pl.ds(tid * 640, 640)])


@jax.jit
def _edge_stats(dst, ts):
    mesh = plsc.VectorSubcoreMesh(core_axis_name="c", subcore_axis_name="s")
    return pl.kernel(
        _edge_stats_body,
        out_type=[jax.ShapeDtypeStruct((NP,), jnp.float32),
                  jax.ShapeDtypeStruct((NP,), jnp.float32)],
        mesh=mesh,
        scratch_types=[
            pltpu.VMEM((EPT,), jnp.int32),
            pltpu.VMEM((EPT,), jnp.float32),
            pltpu.VMEM((NP,), jnp.float32),
            pltpu.VMEM((NTILE, 640), jnp.float32),
            pltpu.VMEM((640,), jnp.float32),
            pltpu.VMEM_SHARED((NTILE, NP), jnp.float32),
        ],
        compiler_params=pltpu.CompilerParams(needs_layout_passes=False),
    )(dst, ts)


def _agg_body(src_hbm, dst_hbm, y_hbm, agg_out,
              src_v, dst_v, rows0, rows1, zero_v, acc_sh,
              sem0, sem1):
    cid = lax.axis_index("c")
    tid = lax.axis_index("s")

    pltpu.sync_copy(src_hbm.at[pl.ds(tid * EPT, EPT)], src_v)
    pltpu.sync_copy(dst_hbm.at[pl.ds(tid * EPT, EPT)], dst_v)
    _zero_vmem_2d(zero_v, ZROWS, CHUNK)

    def process(j, this_rows, this_sem, issue_j, next_rows, next_sem):
        if issue_j is not None:
            pltpu.async_copy(y_hbm.at[src_v.at[pl.ds(issue_j * BE, BE)]],
                             next_rows, next_sem)
        pltpu.make_async_copy(y_hbm.at[src_v.at[pl.ds(j * BE, BE)]],
                              this_rows, this_sem).wait()
        pltpu.sync_copy(this_rows, acc_sh.at[dst_v.at[pl.ds(j * BE, BE)]],
                        add=True)

    for ci in range(NCH // 2):
        def gbody(r, _):
            for k in range(BE // 16):
                off = r * BE + k * 16
                v = src_v[pl.ds(off, 16)]
                if ci == 0:
                    v = v + cid * (NCH // 2) * N
                else:
                    v = v + N
                src_v[pl.ds(off, 16)] = v
            return 0
        lax.fori_loop(0, NBLK, gbody, 0)

        for z in range(NROWT // ZROWS):
            pltpu.sync_copy(
                zero_v, acc_sh.at[pl.ds(tid * NROWT + z * ZROWS, ZROWS)])
        plsc.subcore_barrier()

        pltpu.async_copy(y_hbm.at[src_v.at[pl.ds(0, BE)]], rows0, sem0)

        def blk(jj, _):
            j0 = jj * 2
            process(j0, rows0, sem0, j0 + 1, rows1, sem1)
            process(j0 + 1, rows1, sem1, j0 + 2, rows0, sem0)
            return 0

        lax.fori_loop(0, (NBLK - 1) // 2, blk, 0)
        process(NBLK - 1, rows0, sem0, None, None, None)

        plsc.subcore_barrier()

        c = cid * (NCH // 2) + ci

        @pl.when(tid < NTILE - 1)
        def _():
            pltpu.sync_copy(acc_sh.at[pl.ds(tid * NROWT, NROWT)],
                            agg_out.at[c, pl.ds(tid * NROWT, NROWT)])

        @pl.when(tid == NTILE - 1)
        def _():
            pltpu.sync_copy(acc_sh.at[pl.ds((NTILE - 1) * NROWT, NLAST)],
                            agg_out.at[c, pl.ds((NTILE - 1) * NROWT, NLAST)])

        plsc.subcore_barrier()


@jax.jit
def _aggregate(src2, dst2, y2):
    mesh = plsc.VectorSubcoreMesh(core_axis_name="c", subcore_axis_name="s")
    return pl.kernel(
        _agg_body,
        out_type=jax.ShapeDtypeStruct((NCH, N, CHUNK), jnp.float32),
        mesh=mesh,
        scratch_types=[
            pltpu.VMEM((EPT,), jnp.int32),
            pltpu.VMEM((EPT,), jnp.int32),
            pltpu.VMEM((BE, CHUNK), jnp.float32),
            pltpu.VMEM((BE, CHUNK), jnp.float32),
            pltpu.VMEM((ZROWS, CHUNK), jnp.float32),
            pltpu.VMEM_SHARED((NP, CHUNK), jnp.float32),
            pltpu.SemaphoreType.DMA,
            pltpu.SemaphoreType.DMA,
        ],
        compiler_params=pltpu.CompilerParams(
            needs_layout_passes=False, use_tc_tiling_on_sc=False),
    )(src2, dst2, y2)


def _tc_a_body(x_ref, win_ref, bin_ref, wt_ref, bt_ref, ntc_ref, ntf_ref,
               deg_ref, wg_ref, y_ref):
    h = jnp.dot(x_ref[...], win_ref[...],
                preferred_element_type=jnp.float32) + bin_ref[...]
    ntf = ntf_ref[...]
    tmin = jnp.min(ntf)
    tmax = jnp.max(ntf)
    nt = ntc_ref[...]
    ntn = jnp.where(tmax > tmin, (nt - tmin) / (tmax - tmin + 1e-8), nt)
    h = h + ntn * wt_ref[...] + bt_ref[...]
    h = jnp.maximum(h, 0.0)
    dinv = lax.rsqrt(deg_ref[...] + 1.0)
    yb = jnp.dot(h, wg_ref[...], preferred_element_type=jnp.float32) * dinv
    for c in range(NCH):
        y_ref[c] = yb[:, c * CHUNK:(c + 1) * CHUNK]


@jax.jit
def _tc_a(x, W_in, b_in, W_t, b_t, nt_col, nt_full, deg_col, W_gcn):
    blk = 400
    grid = N // blk
    return pl.pallas_call(
        _tc_a_body,
        grid=(grid,),
        in_specs=[
            pl.BlockSpec((blk, FIN), lambda i: (i, 0)),
            pl.BlockSpec((FIN, H), lambda i: (0, 0)),
            pl.BlockSpec((1, H), lambda i: (0, 0)),
            pl.BlockSpec((1, H), lambda i: (0, 0)),
            pl.BlockSpec((1, H), lambda i: (0, 0)),
            pl.BlockSpec((blk, 1), lambda i: (i, 0)),
            pl.BlockSpec((8, 1250), lambda i: (0, 0)),
            pl.BlockSpec((blk, 1), lambda i: (i, 0)),
            pl.BlockSpec((H, H), lambda i: (0, 0)),
        ],
        out_specs=pl.BlockSpec((NCH, blk, CHUNK), lambda i: (0, i, 0)),
        out_shape=jax.ShapeDtypeStruct((NCH, N, CHUNK), jnp.float32),
    )(x, W_in, b_in, W_t, b_t, nt_col, nt_full, deg_col, W_gcn)


def _tc_b_body(agg_ref, y_ref, deg_ref, bgcn_ref, wc1_ref, bc1_ref,
               wc2_ref, bc2_ref, out_ref):
    a = agg_ref[...]
    aggc = jnp.concatenate([a[c] for c in range(NCH)], axis=-1)
    yv = y_ref[...]
    yc = jnp.concatenate([yv[c] for c in range(NCH)], axis=-1)
    dinv = lax.rsqrt(deg_ref[...] + 1.0)
    h2 = (aggc + yc) * dinv + bgcn_ref[...]
    h1 = jnp.maximum(
        jnp.dot(h2, wc1_ref[...], preferred_element_type=jnp.float32)
        + bc1_ref[...], 0.0)
    out_ref[...] = jnp.dot(h1, wc2_ref[...],
                           preferred_element_type=jnp.float32) + bc2_ref[...]


@jax.jit
def _tc_b(agg, y, deg_col, b_gcn, W_c1, b_c1, W_c2, b_c2):
    blk = 400
    grid = N // blk
    return pl.pallas_call(
        _tc_b_body,
        grid=(grid,),
        in_specs=[
            pl.BlockSpec((NCH, blk, CHUNK), lambda i: (0, i, 0)),
            pl.BlockSpec((NCH, blk, CHUNK), lambda i: (0, i, 0)),
            pl.BlockSpec((blk, 1), lambda i: (i, 0)),
            pl.BlockSpec((1, H), lambda i: (0, 0)),
            pl.BlockSpec((H, 64), lambda i: (0, 0)),
            pl.BlockSpec((1, 64), lambda i: (0, 0)),
            pl.BlockSpec((64, 2), lambda i: (0, 0)),
            pl.BlockSpec((1, 2), lambda i: (0, 0)),
        ],
        out_specs=pl.BlockSpec((blk, 2), lambda i: (i, 0)),
        out_shape=jax.ShapeDtypeStruct((N, 2), jnp.float32),
    )(agg, y, deg_col, b_gcn, W_c1, b_c1, W_c2, b_c2)


def kernel(x, edge_index, timestamps, W_in, b_in, W_t, b_t, W_gcn, b_gcn,
           W_c1, b_c1, W_c2, b_c2):
    src = edge_index[0]
    dst = edge_index[1]
    nt_p, deg_p = _edge_stats(dst, timestamps)
    nt = nt_p[:N]
    deg_in = deg_p[:N]
    y = _tc_a(x, W_in, b_in.reshape(1, H), W_t, b_t.reshape(1, H),
              nt.reshape(N, 1), nt.reshape(8, 1250), deg_in.reshape(N, 1),
              W_gcn)
    agg = _aggregate(src, dst, y.reshape(NCH * N, CHUNK))
    return _tc_b(agg, y, deg_in.reshape(N, 1), b_gcn.reshape(1, H),
                 W_c1, b_c1.reshape(1, 64), W_c2, b_c2.reshape(1, 2))
